# baseline (speedup 1.0000x reference)
import functools

import jax
import jax.numpy as jnp
from jax import lax
from jax.experimental import pallas as pl
from jax.experimental.pallas import tpu as pltpu
from jax.experimental.pallas import tpu_sc as plsc

_HIDDEN = 128
_LANES = 16
_GROUPS = _HIDDEN // _LANES
_EPS = 1e-12


_GATHER_DN = lax.GatherDimensionNumbers(
    offset_dims=(), collapsed_slice_dims=(0,), start_index_map=(0,))


def _all_sum(x):
    for k in (1, 2, 4, 8):
        idx = (lax.iota(jnp.int32, _LANES) ^ k)[:, None]
        x = x + lax.gather(x, idx, _GATHER_DN, (1,),
                           mode=lax.GatherScatterMode.PROMISE_IN_BOUNDS)
    return x


def _rsqrt(v):
    i = lax.bitcast_convert_type(v, jnp.int32)
    y = lax.bitcast_convert_type(jnp.int32(0x5F3759DF) - (i >> 1), jnp.float32)
    for _ in range(3):
        y = y * (1.5 - 0.5 * v * y * y)
    return y


def _make_sc_kernel(n_tok, seq_len):
    info = plsc.get_sparse_core_info()
    nc, ns = info.num_cores, info.num_subcores
    nw = nc * ns
    tpw = n_tok // nw
    n_chunks = tpw // 128
    mesh = plsc.VectorSubcoreMesh(core_axis_name="c", subcore_axis_name="s")

    @functools.partial(
        pl.kernel,
        out_type=jax.ShapeDtypeStruct((n_tok, _HIDDEN), jnp.float32),
        mesh=mesh,
        scratch_types=[
            pltpu.VMEM((n_chunks, 128), jnp.int32),
            pltpu.VMEM((n_chunks, 128), jnp.int32),
            pltpu.VMEM((tpw, _HIDDEN), jnp.float32),
            pltpu.VMEM((tpw, _HIDDEN), jnp.float32),
            pltpu.VMEM((tpw, _HIDDEN), jnp.float32),
            pltpu.VMEM((_HIDDEN,), jnp.float32),
            pltpu.VMEM((_HIDDEN,), jnp.float32),
            pltpu.SemaphoreType.DMA,
        ],
    )
    def sc_kernel(ids_hbm, tt_hbm, word_hbm, pos_hbm, type_hbm, gam_hbm,
                  bet_hbm, out_hbm, idx_v, tidx_v, rows_v, pos_v, trow_v,
                  gam_v, bet_v, sem):
        wid = lax.axis_index("s") * nc + lax.axis_index("c")
        base = wid * tpw

        pltpu.sync_copy(ids_hbm.at[pl.ds(wid * n_chunks, n_chunks)], idx_v)
        pltpu.sync_copy(tt_hbm.at[pl.ds(wid * n_chunks, n_chunks)], tidx_v)
        copies = []
        for j in range(n_chunks):
            copies.append(pltpu.async_copy(
                word_hbm.at[idx_v.at[j]],
                rows_v.at[pl.ds(j * 128, 128)], sem))
            copies.append(pltpu.async_copy(
                type_hbm.at[tidx_v.at[j]],
                trow_v.at[pl.ds(j * 128, 128)], sem))
        pos_base = base % seq_len
        pltpu.sync_copy(pos_hbm.at[pl.ds(pos_base, tpw)], pos_v)
        pltpu.sync_copy(gam_hbm, gam_v)
        pltpu.sync_copy(bet_hbm, bet_v)

        gam = [gam_v[pl.ds(g * _LANES, _LANES)] for g in range(_GROUPS)]
        bet = [bet_v[pl.ds(g * _LANES, _LANES)] for g in range(_GROUPS)]

        for c in copies:
            c.wait()

        def body(t, carry):
            acc = jnp.zeros((_LANES,), jnp.float32)
            accsq = jnp.zeros((_LANES,), jnp.float32)
            xs = []
            for g in range(_GROUPS):
                sl = pl.ds(g * _LANES, _LANES)
                x = rows_v[t, sl] + pos_v[t, sl] + trow_v[t, sl]
                xs.append(x)
                acc = acc + x
                accsq = accsq + x * x
            mu = _all_sum(acc) * (1.0 / _HIDDEN)
            ex2 = _all_sum(accsq) * (1.0 / _HIDDEN)
            r = _rsqrt(ex2 - mu * mu + _EPS)
            for g in range(_GROUPS):
                sl = pl.ds(g * _LANES, _LANES)
                rows_v[t, sl] = (xs[g] - mu) * r * gam[g] + bet[g]
            return carry

        lax.fori_loop(0, tpw, body, 0)
        pltpu.sync_copy(rows_v, out_hbm.at[pl.ds(base, tpw)])

    return sc_kernel


@jax.jit
def kernel(input_ids, token_type_ids, word_emb, pos_emb, type_emb, ln_gamma,
           ln_beta):
    b, s = input_ids.shape
    n_tok = b * s
    ids = input_ids.astype(jnp.int32).reshape(n_tok // 128, 128)
    tts = token_type_ids.astype(jnp.int32).reshape(n_tok // 128, 128)
    out = _make_sc_kernel(n_tok, s)(
        ids, tts, word_emb, pos_emb, type_emb, ln_gamma, ln_beta)
    return out.reshape(b, s, _HIDDEN)
